# baseline (speedup 1.0000x reference)
import functools

import jax
import jax.numpy as jnp
import numpy as np
from jax import lax
from jax.experimental import pallas as pl
from jax.experimental.pallas import tpu as pltpu
from jax.experimental.pallas import tpu_sc as plsc

NC = 2
NS = 16
NW = NC * NS
CH = 80
EPS = 1e-5
F32 = jnp.float32

_SC_MESH = plsc.VectorSubcoreMesh(
    core_axis_name="c", subcore_axis_name="s", num_cores=NC, num_subcores=NS)


def _edge_w_body(e_ref, W_ref, b_ref, o_ref):
    o_ref[...] = (jnp.dot(e_ref[...], W_ref[...], preferred_element_type=F32)
                  + b_ref[...])


def _ab_body(x_ref, Wa_ref, Wb_ref, a_ref, b_ref):
    a_ref[...] = jnp.dot(x_ref[...], Wa_ref[...], preferred_element_type=F32)
    b_ref[...] = jnp.dot(x_ref[...], Wb_ref[...], preferred_element_type=F32)


def _mid_body(M, y1_ref, mom_ref, gbt1_ref, W2_ref, b2_ref, gbt2_ref,
              y2_ref, ac2_ref, a1c1_ref, macc_ref):
    D = W2_ref.shape[0] // 2

    @pl.when(pl.program_id(0) == 0)
    def _init():
        msum = jnp.sum(mom_ref[...], axis=0)
        mean1 = msum[0:1] / M
        var1 = msum[1:2] / M - mean1 * mean1
        a1 = gbt1_ref[0:1] * jax.lax.rsqrt(var1 + EPS)
        a1c1_ref[0:1] = a1
        a1c1_ref[1:2] = gbt1_ref[1:2] - a1 * mean1
        macc_ref[...] = jnp.zeros_like(macc_ref)
        ac2_ref[...] = jnp.zeros_like(ac2_ref)

    y1f = y1_ref[...].astype(F32)
    h = jnp.maximum(y1f * a1c1_ref[0:1] + a1c1_ref[1:2], 0.0)
    hb = h.astype(jnp.bfloat16)
    y2 = jnp.dot(hb, W2_ref[...], preferred_element_type=F32) + b2_ref[...]
    y2p = y2[:, :D]
    y2c = y2[:, D:]
    y2_ref[0] = y2p
    y2_ref[1] = y2c

    macc_ref[0:1] += jnp.sum(y2p, axis=0, keepdims=True)
    macc_ref[1:2] += jnp.sum(y2p * y2p, axis=0, keepdims=True)
    macc_ref[2:3] += jnp.sum(y2c, axis=0, keepdims=True)
    macc_ref[3:4] += jnp.sum(y2c * y2c, axis=0, keepdims=True)

    @pl.when(pl.program_id(0) == pl.num_programs(0) - 1)
    def _fin():
        mean2 = jnp.stack([macc_ref[0], macc_ref[2]]) / M
        var2 = jnp.stack([macc_ref[1], macc_ref[3]]) / M - mean2 * mean2
        a2 = gbt2_ref[0:2] * jax.lax.rsqrt(var2 + EPS)
        c2 = gbt2_ref[2:4] - a2 * mean2
        ac2_ref[...] = jnp.stack([a2, c2], axis=1)


def _final_body(x_ref, agg_ref, deg_ref, fW_ref, fp_ref, Wa_ref, Wb_ref,
                xo_ref, a_ref, b_ref):
    N, D = x_ref.shape
    op = agg_ref[0, :N]
    oc = agg_ref[1, :N]
    deg_d = deg_ref[0, :N, 0:1]
    deg_s = deg_ref[1, :N, 0:1]
    fi = jnp.where(deg_d > 0, 1.0 / deg_d, 0.0) * op
    fo = jnp.where(deg_s > 0, 1.0 / deg_s, 0.0) * oc
    z = x_ref[...] + fi + fo
    y = jnp.dot(z, fW_ref[...], preferred_element_type=F32) + fp_ref[0:1, :]
    mu = jnp.mean(y, axis=0, keepdims=True)
    var = jnp.mean((y - mu) * (y - mu), axis=0, keepdims=True)
    bn = fp_ref[1:2, :] * (y - mu) / jnp.sqrt(var + EPS) + fp_ref[2:3, :]
    xn = x_ref[...] + jnp.maximum(bn, 0.0)
    xo_ref[...] = xn
    a_ref[...] = jnp.dot(xn, Wa_ref[...], preferred_element_type=F32)
    b_ref[...] = jnp.dot(xn, Wb_ref[...], preferred_element_type=F32)


GCH = 40


def _gather_body(A, B, dst3, src3, w, y1, mom,
                 idxd, idxs, bufA, bufB, bufW, bufY, accv,
                 gsem0, gsem1, osem0, osem1, isem0, isem1):
    nchunk = dst3.shape[1]
    per_w = nchunk * GCH
    c = lax.axis_index("c")
    s = lax.axis_index("s")
    wid = s * NC + c
    base_w = wid * per_w

    bA = (bufA.at[0], bufA.at[1])
    bB = (bufB.at[0], bufB.at[1])
    bW = (bufW.at[0], bufW.at[1])
    bY = (bufY.at[0], bufY.at[1])
    gsem = (gsem0, gsem1)
    osem = (osem0, osem1)
    isem = (isem0, isem1)

    def i_copies(k, slot):
        sem = isem[slot % 2]
        return (
            pltpu.make_async_copy(dst3.at[wid, k, 0], idxd.at[slot], sem),
            pltpu.make_async_copy(src3.at[wid, k, 0], idxs.at[slot], sem),
        )

    def g_copies(k, b, slot):
        base = base_w + k * GCH
        return (
            pltpu.make_async_copy(A.at[idxd.at[slot]], bA[b], gsem[b]),
            pltpu.make_async_copy(B.at[idxs.at[slot]], bB[b], gsem[b]),
            pltpu.make_async_copy(w.at[pl.ds(base, GCH)], bW[b], gsem[b]),
        )

    def o_copy(k, b):
        base = base_w + k * GCH
        return pltpu.make_async_copy(bY[b], y1.at[pl.ds(base, GCH)], osem[b])

    for q in (0, 1):
        for cp in i_copies(q, q):
            cp.start()
        for cp in i_copies(q, q):
            cp.wait()
    for q in (2, 3):
        for cp in i_copies(q, q):
            cp.start()
    for cp in g_copies(0, 0, 0):
        cp.start()
    for cp in g_copies(1, 1, 1):
        cp.start()

    zero16 = jnp.zeros((16,), F32)
    acc0 = (tuple(zero16 for _ in range(16)), tuple(zero16 for _ in range(16)))

    def dstep(k2, acc):
        q2 = k2 % 2
        for b in range(2):
            k = 2 * k2 + b
            for q in range(2):
                @pl.when(q2 == q)
                def _(q=q):
                    for cp in g_copies(k, b, b + 2 * q):
                        cp.wait()

            @pl.when(k >= 2)
            def _():
                o_copy(k - 2, b).wait()

            def row(r, acc_r):
                ss, qq = acc_r
                nss = list(ss)
                nqq = list(qq)
                for g in range(8):
                    sl0 = pl.ds(32 * g, 16)
                    sl1 = pl.ds(32 * g + 16, 16)
                    sl32 = pl.ds(32 * g, 32)
                    y0 = bA[b][r, sl0] + bB[b][r, sl0] + bW[b][r, sl0]
                    y1v = bA[b][r, sl1] + bB[b][r, sl1] + bW[b][r, sl1]
                    bY[b][r, :, pl.ds(16 * g, 16)] = jnp.stack(
                        [y0.astype(jnp.bfloat16), y1v.astype(jnp.bfloat16)])
                    nss[2 * g] = nss[2 * g] + y0
                    nqq[2 * g] = nqq[2 * g] + y0 * y0
                    nss[2 * g + 1] = nss[2 * g + 1] + y1v
                    nqq[2 * g + 1] = nqq[2 * g + 1] + y1v * y1v
                return (tuple(nss), tuple(nqq))

            acc = lax.fori_loop(0, GCH, row, acc)
            o_copy(k, b).start()

            for q in range(2):
                @pl.when((k + 2 < nchunk) & (q2 == q))
                def _(q=q):
                    slot = b + 2 * (1 - q)
                    for cp in i_copies(k + 2, slot):
                        cp.wait()
                    for cp in g_copies(k + 2, b, slot):
                        cp.start()

                @pl.when((k + 4 < nchunk) & (q2 == q))
                def _(q=q):
                    for cp in i_copies(k + 4, b + 2 * q):
                        cp.start()
        return acc

    ss, qq = lax.fori_loop(0, nchunk // 2, dstep, acc0)
    o_copy(nchunk - 2, 0).wait()
    o_copy(nchunk - 1, 1).wait()
    for g in range(16):
        sl = pl.ds(g * 16, 16)
        accv[0, sl] = ss[g]
        accv[1, sl] = qq[g]
    pltpu.sync_copy(accv, mom.at[wid])


def _zero_spmem(zbuf, accS, s, rows_per_s, ncols):
    zrows = zbuf.shape[0]

    def zrow(r, _):
        for g in range(ncols // 16):
            zbuf[r, pl.ds(g * 16, 16)] = jnp.zeros((16,), F32)
        return 0
    lax.fori_loop(0, zrows, zrow, 0)
    for t in range(rows_per_s // zrows):
        pltpu.sync_copy(zbuf, accS.at[pl.ds(s * rows_per_s + t * zrows, zrows)])


def _copy_idx(dsta, srca, base, idxb, c):
    @pl.when(c == 0)
    def _():
        pltpu.sync_copy(dsta.at[pl.ds(base, CH)], idxb)

    @pl.when(c == 1)
    def _():
        pltpu.sync_copy(srca.at[pl.ds(base, CH)], idxb)


def _deg_body(ridx, out, idxc, oneb, zbuf, accS,
              isem0, isem1, ssem0, ssem1):
    NP = out.shape[1]
    nchunk = ridx.shape[2]
    rows_per_s = NP // NS
    c = lax.axis_index("c")
    s = lax.axis_index("s")
    isem = (isem0, isem1)
    ssem = (ssem0, ssem1)

    _zero_spmem(zbuf, accS, s, rows_per_s, 128)

    lane = lax.iota(jnp.int32, 16)
    onehot = jnp.where(lane == 0, jnp.float32(1.0), jnp.float32(0.0))
    zero16 = jnp.zeros((16,), F32)

    def prow(r, _):
        oneb[r, pl.ds(0, 16)] = onehot
        for g in range(1, 8):
            oneb[r, pl.ds(g * 16, 16)] = zero16
        return 0
    lax.fori_loop(0, CH, prow, 0)
    plsc.subcore_barrier()

    def i_copy(k, slot):
        return pltpu.make_async_copy(ridx.at[c, s, k, 0], idxc.at[slot],
                                     isem[slot % 2])

    def s_copy(k, slot):
        return pltpu.make_async_copy(oneb, accS.at[idxc.at[slot]],
                                     ssem[slot % 2])

    for q in (0, 1):
        i_copy(q, q).start()

    def dstep(k2, _):
        q2 = k2 % 2
        for b in range(2):
            k = 2 * k2 + b
            for q in range(2):
                @pl.when(q2 == q)
                def _(q=q):
                    slot = b + 2 * q
                    other = b + 2 * (1 - q)
                    i_copy(k, slot).wait()

                    @pl.when(k >= 2)
                    def _():
                        s_copy(k - 2, other).wait()

                    s_copy(k, slot).start(add=True)

                    @pl.when(k + 2 < nchunk)
                    def _():
                        i_copy(k + 2, other).start()
        return 0

    lax.fori_loop(0, nchunk // 2, dstep, 0)
    s_copy(nchunk - 2, (nchunk - 2) % 4).wait()
    s_copy(nchunk - 1, (nchunk - 1) % 4).wait()
    plsc.subcore_barrier()
    pltpu.sync_copy(accS.at[pl.ds(s * rows_per_s, rows_per_s)],
                    out.at[c, pl.ds(s * rows_per_s, rows_per_s)])


def _scatter_body(y2, ridx, ac2, out,
                  idxc, y2b, msgb, zbuf, acv, accS,
                  ysem0, ysem1, ssem0, ssem1):
    NP = out.shape[1]
    nchunk = ridx.shape[2]
    per_s = nchunk * CH
    rows_per_s = NP // NS
    c = lax.axis_index("c")
    s = lax.axis_index("s")

    yb = (y2b.at[0], y2b.at[1])
    mb = (msgb.at[0], msgb.at[1])
    ysem = (ysem0, ysem1)
    ssem = (ssem0, ssem1)

    pltpu.sync_copy(ac2.at[c], acv)
    _zero_spmem(zbuf, accS, s, rows_per_s, 128)
    plsc.subcore_barrier()

    def y_copies(k, b, slot):
        base = s * per_s + k * CH
        return (
            pltpu.make_async_copy(y2.at[c, pl.ds(base, CH)], yb[b], ysem[b]),
            pltpu.make_async_copy(ridx.at[c, s, k, 0], idxc.at[slot], ysem[b]),
        )

    def s_copy(k, b, slot):
        return pltpu.make_async_copy(mb[b], accS.at[idxc.at[slot]], ssem[b])

    for q in (0, 1):
        for cp in y_copies(q, q, q):
            cp.start()

    def dstep(k2, _):
        q2 = k2 % 2
        for b in range(2):
            k = 2 * k2 + b
            for q in range(2):
                @pl.when(q2 == q)
                def _(q=q):
                    for cp in y_copies(k, b, b + 2 * q):
                        cp.wait()

                @pl.when((k >= 2) & (q2 == q))
                def _(q=q):
                    s_copy(k - 2, b, b + 2 * (1 - q)).wait()

            def row(r, __):
                for g in range(8):
                    sl = pl.ds(g * 16, 16)
                    v = acv[0, sl] * yb[b][r, sl] + acv[1, sl]
                    mb[b][r, sl] = jnp.maximum(v, 0.0)
                return 0
            lax.fori_loop(0, CH, row, 0)

            for q in range(2):
                @pl.when(q2 == q)
                def _(q=q):
                    s_copy(k, b, b + 2 * q).start(add=True)

                @pl.when((k + 2 < nchunk) & (q2 == q))
                def _(q=q):
                    for cp in y_copies(k + 2, b, b + 2 * (1 - q)):
                        cp.start()
        return 0

    lax.fori_loop(0, nchunk // 2, dstep, 0)
    s_copy(nchunk - 2, 0, (nchunk - 2) % 4).wait()
    s_copy(nchunk - 1, 1, (nchunk - 1) % 4).wait()
    plsc.subcore_barrier()
    pltpu.sync_copy(accS.at[pl.ds(s * rows_per_s, rows_per_s)],
                    out.at[c, pl.ds(s * rows_per_s, rows_per_s)])


def kernel(nodes, edges, edge_attr,
           pW1, pb1, pg1, pbt1, pW2, pb2, pg2, pbt2,
           cW1, cb1, cg1, cbt1, cW2, cb2, cg2, cbt2,
           fW, fb, fg, fbt):
    N, D = nodes.shape
    M = edges.shape[1]
    E = edge_attr.shape[1]
    D2 = 2 * D

    Wa = jnp.concatenate([pW1[:D], cW1[D:D2]], axis=1)
    Wb = jnp.concatenate([pW1[D:D2], cW1[:D]], axis=1)
    We = jnp.concatenate([pW1[D2:], cW1[D2:]], axis=1)
    be = jnp.concatenate([pb1, cb1]).reshape(1, D2)
    g1v = jnp.concatenate([pg1, cg1])
    bt1v = jnp.concatenate([pbt1, cbt1])
    b2s = jnp.stack([pb2, cb2])
    g2s = jnp.stack([pg2, cg2])
    bt2s = jnp.stack([pbt2, cbt2])
    fp = jnp.stack([fb, fg, fbt])

    BLK_E = 2000
    w = pl.pallas_call(
        _edge_w_body,
        grid=(M // BLK_E,),
        in_specs=[
            pl.BlockSpec((BLK_E, E), lambda i: (i, 0)),
            pl.BlockSpec((E, D2), lambda i: (0, 0)),
            pl.BlockSpec((1, D2), lambda i: (0, 0)),
        ],
        out_specs=pl.BlockSpec((BLK_E, D2), lambda i: (i, 0)),
        out_shape=jax.ShapeDtypeStruct((M, D2), F32),
    )(edge_attr, We, be)

    ab_call = pl.pallas_call(
        _ab_body,
        out_shape=(jax.ShapeDtypeStruct((N, D2), F32),
                   jax.ShapeDtypeStruct((N, D2), F32)),
    )

    GN = M // NW // GCH
    gather_call = pl.kernel(
        _gather_body,
        out_type=(jax.ShapeDtypeStruct((M, 2, D2 // 2), jnp.bfloat16),
                  jax.ShapeDtypeStruct((NW, 2, D2), F32)),
        mesh=_SC_MESH,
        scratch_types=[
            pltpu.VMEM((4, GCH), jnp.int32),
            pltpu.VMEM((4, GCH), jnp.int32),
            pltpu.VMEM((2, GCH, D2), F32),
            pltpu.VMEM((2, GCH, D2), F32),
            pltpu.VMEM((2, GCH, D2), F32),
            pltpu.VMEM((2, GCH, 2, D2 // 2), jnp.bfloat16),
            pltpu.VMEM((2, D2), F32),
            pltpu.SemaphoreType.DMA,
            pltpu.SemaphoreType.DMA,
            pltpu.SemaphoreType.DMA,
            pltpu.SemaphoreType.DMA,
            pltpu.SemaphoreType.DMA,
            pltpu.SemaphoreType.DMA,
        ],
    )

    BLK_M = 512
    mid_call = pl.pallas_call(
        functools.partial(_mid_body, M),
        grid=(M // BLK_M,),
        in_specs=[
            pl.BlockSpec((BLK_M, D2), lambda i: (i, 0)),
            pl.BlockSpec((NW, 2, D2), lambda i: (0, 0, 0)),
            pl.BlockSpec((2, D2), lambda i: (0, 0)),
            pl.BlockSpec((D2, D2), lambda i: (0, 0)),
            pl.BlockSpec((1, D2), lambda i: (0, 0)),
            pl.BlockSpec((4, D), lambda i: (0, 0)),
        ],
        out_specs=(pl.BlockSpec((2, BLK_M, D), lambda i: (0, i, 0)),
                   pl.BlockSpec((2, 2, D), lambda i: (0, 0, 0))),
        out_shape=(jax.ShapeDtypeStruct((2, M, D), F32),
                   jax.ShapeDtypeStruct((2, 2, D), F32)),
        scratch_shapes=[pltpu.VMEM((2, D2), F32), pltpu.VMEM((4, D), F32)],
    )

    NP = ((N + 128 * NS - 1) // (128 * NS)) * (128 * NS)
    SN = M // NS // CH
    scatter_call = pl.kernel(
        _scatter_body,
        out_type=jax.ShapeDtypeStruct((2, NP, D), F32),
        mesh=_SC_MESH,
        scratch_types=[
            pltpu.VMEM((4, CH), jnp.int32),
            pltpu.VMEM((2, CH, D), F32),
            pltpu.VMEM((2, CH, D), F32),
            pltpu.VMEM((32, D), F32),
            pltpu.VMEM((2, D), F32),
            pltpu.VMEM_SHARED((NP, D), F32),
            pltpu.SemaphoreType.DMA,
            pltpu.SemaphoreType.DMA,
            pltpu.SemaphoreType.DMA,
            pltpu.SemaphoreType.DMA,
        ],
    )

    deg_call = pl.kernel(
        _deg_body,
        out_type=jax.ShapeDtypeStruct((2, NP, D), F32),
        mesh=_SC_MESH,
        scratch_types=[
            pltpu.VMEM((4, CH), jnp.int32),
            pltpu.VMEM((CH, D), F32),
            pltpu.VMEM((128, D), F32),
            pltpu.VMEM_SHARED((NP, D), F32),
            pltpu.SemaphoreType.DMA,
            pltpu.SemaphoreType.DMA,
            pltpu.SemaphoreType.DMA,
            pltpu.SemaphoreType.DMA,
        ],
    )

    final_call = pl.pallas_call(
        _final_body,
        out_shape=(jax.ShapeDtypeStruct((N, D), F32),
                   jax.ShapeDtypeStruct((N, D2), F32),
                   jax.ShapeDtypeStruct((N, D2), F32)),
    )

    Qp = np.empty(D2, np.int32)
    for k_ in range(D2):
        h_, m_ = divmod(k_, D2 // 2)
        Qp[k_] = 32 * (m_ // 16) + 16 * h_ + (m_ % 16)
    gbt1 = jnp.stack([g1v, bt1v])[:, Qp]
    gbt2 = jnp.concatenate([g2s, bt2s])
    W2L = jnp.zeros((D2, D2), F32)
    W2L = W2L.at[:D, :D].set(pW2).at[D:, D:].set(cW2)
    W2c = W2L[Qp, :].astype(jnp.bfloat16)
    b2c = jnp.concatenate([pb2, cb2]).reshape(1, D2)
    dsta = edges[1]
    srca = edges[0]
    dst3g = dsta.reshape(NW, GN, 1, GCH)
    src3g = srca.reshape(NW, GN, 1, GCH)
    ridx = jnp.stack([dsta, srca]).reshape(2, NS, SN, 1, CH)
    degs = deg_call(ridx)
    x = nodes
    A, B = ab_call(x, Wa, Wb)
    for _ in range(2):
        y1, mom = gather_call(A, B, dst3g, src3g, w)
        mom_p = mom[:, :, Qp]
        y2, ac2 = mid_call(y1.reshape(M, D2), mom_p, gbt1, W2c, b2c, gbt2)
        agg = scatter_call(y2, ridx, ac2)
        x, A, B = final_call(x, agg, degs, fW, fp, Wa, Wb)
    return x


# speedup vs baseline: 1.2394x; 1.2394x over previous
import functools

import jax
import jax.numpy as jnp
from jax import lax
from jax.experimental import pallas as pl
from jax.experimental.pallas import tpu as pltpu
from jax.experimental.pallas import tpu_sc as plsc

NC = 2
NS = 16
NW = NC * NS
CH = 80
EPS = 1e-5
F32 = jnp.float32

_SC_MESH = plsc.VectorSubcoreMesh(
    core_axis_name="c", subcore_axis_name="s", num_cores=NC, num_subcores=NS)


def _edge_w_body(e_ref, W_ref, b_ref, o_ref):
    o_ref[...] = (
        jnp.dot(e_ref[...], W_ref[...], preferred_element_type=F32)
        + b_ref[...])


def _ab_body(x_ref, Wa_ref, Wb_ref, a_ref, b_ref):
    a_ref[...] = jnp.dot(x_ref[...], Wa_ref[...], preferred_element_type=F32)
    b_ref[...] = jnp.dot(x_ref[...], Wb_ref[...], preferred_element_type=F32)


def _mid_body(M, y1_ref, mom_ref, gbt1_ref, Wp_ref, Wc_ref, b2_ref, gbt2_ref,
              y2_ref, ac2_ref, a1c1_ref, macc_ref):
    D = Wp_ref.shape[0]

    @pl.when(pl.program_id(0) == 0)
    def _init():
        msum = jnp.sum(mom_ref[...], axis=0)
        mean1 = msum[0:1] / M
        var1 = msum[1:2] / M - mean1 * mean1
        a1 = gbt1_ref[0:1] * jax.lax.rsqrt(var1 + EPS)
        a1c1_ref[0:1] = a1
        a1c1_ref[1:2] = gbt1_ref[1:2] - a1 * mean1
        macc_ref[...] = jnp.zeros_like(macc_ref)
        ac2_ref[...] = jnp.zeros_like(ac2_ref)

    h = jnp.maximum(y1_ref[...] * a1c1_ref[0:1] + a1c1_ref[1:2], 0.0)
    hb = h.astype(jnp.bfloat16)
    y2p = jnp.dot(hb[:, :D], Wp_ref[...], preferred_element_type=F32) + b2_ref[0:1, :]
    y2c = jnp.dot(hb[:, D:], Wc_ref[...], preferred_element_type=F32) + b2_ref[1:2, :]
    y2_ref[0] = y2p
    y2_ref[1] = y2c

    macc_ref[0:1] += jnp.sum(y2p, axis=0, keepdims=True)
    macc_ref[1:2] += jnp.sum(y2p * y2p, axis=0, keepdims=True)
    macc_ref[2:3] += jnp.sum(y2c, axis=0, keepdims=True)
    macc_ref[3:4] += jnp.sum(y2c * y2c, axis=0, keepdims=True)

    @pl.when(pl.program_id(0) == pl.num_programs(0) - 1)
    def _fin():
        mean2 = jnp.stack([macc_ref[0], macc_ref[2]]) / M
        var2 = jnp.stack([macc_ref[1], macc_ref[3]]) / M - mean2 * mean2
        a2 = gbt2_ref[0:2] * jax.lax.rsqrt(var2 + EPS)
        c2 = gbt2_ref[2:4] - a2 * mean2
        ac2_ref[...] = jnp.stack([a2, c2], axis=1)


def _final_body(x_ref, agg_ref, deg_ref, fW_ref, fp_ref, Wa_ref, Wb_ref,
                xo_ref, a_ref, b_ref):
    N, D = x_ref.shape
    op = agg_ref[0, :N]
    oc = agg_ref[1, :N]
    deg_d = deg_ref[0, :N, 0:1]
    deg_s = deg_ref[1, :N, 0:1]
    fi = jnp.where(deg_d > 0, 1.0 / deg_d, 0.0) * op
    fo = jnp.where(deg_s > 0, 1.0 / deg_s, 0.0) * oc
    z = x_ref[...] + fi + fo
    y = jnp.dot(z, fW_ref[...], preferred_element_type=F32) + fp_ref[0:1, :]
    mu = jnp.mean(y, axis=0, keepdims=True)
    var = jnp.mean((y - mu) * (y - mu), axis=0, keepdims=True)
    bn = fp_ref[1:2, :] * (y - mu) / jnp.sqrt(var + EPS) + fp_ref[2:3, :]
    xn = x_ref[...] + jnp.maximum(bn, 0.0)
    xo_ref[...] = xn
    a_ref[...] = jnp.dot(xn, Wa_ref[...], preferred_element_type=F32)
    b_ref[...] = jnp.dot(xn, Wb_ref[...], preferred_element_type=F32)


GCH = 40


def _gather_body(A, B, dst3, src3, w, y1, mom,
                 idxd, idxs, bufA, bufB, bufW, bufY, accv,
                 gsem0, gsem1, osem0, osem1, isem0, isem1):
    nchunk = dst3.shape[1]
    per_w = nchunk * GCH
    c = lax.axis_index("c")
    s = lax.axis_index("s")
    wid = s * NC + c
    base_w = wid * per_w

    bA = (bufA.at[0], bufA.at[1])
    bB = (bufB.at[0], bufB.at[1])
    bW = (bufW.at[0], bufW.at[1])
    bY = (bufY.at[0], bufY.at[1])
    gsem = (gsem0, gsem1)
    osem = (osem0, osem1)
    isem = (isem0, isem1)

    def i_copies(k, slot):
        sem = isem[slot % 2]
        return (
            pltpu.make_async_copy(dst3.at[wid, k, 0], idxd.at[slot], sem),
            pltpu.make_async_copy(src3.at[wid, k, 0], idxs.at[slot], sem),
        )

    def g_copies(k, b, slot):
        base = base_w + k * GCH
        return (
            pltpu.make_async_copy(A.at[idxd.at[slot]], bA[b], gsem[b]),
            pltpu.make_async_copy(B.at[idxs.at[slot]], bB[b], gsem[b]),
            pltpu.make_async_copy(w.at[pl.ds(base, GCH)], bW[b], gsem[b]),
        )

    def o_copy(k, b):
        base = base_w + k * GCH
        return pltpu.make_async_copy(bY[b], y1.at[pl.ds(base, GCH)], osem[b])

    for q in (0, 1):
        for cp in i_copies(q, q):
            cp.start()
        for cp in i_copies(q, q):
            cp.wait()
    for q in (2, 3):
        for cp in i_copies(q, q):
            cp.start()
    for cp in g_copies(0, 0, 0):
        cp.start()
    for cp in g_copies(1, 1, 1):
        cp.start()

    zero16 = jnp.zeros((16,), F32)
    acc0 = (tuple(zero16 for _ in range(16)), tuple(zero16 for _ in range(16)))

    def dstep(k2, acc):
        q2 = k2 % 2
        for b in range(2):
            k = 2 * k2 + b
            for q in range(2):
                @pl.when(q2 == q)
                def _(q=q):
                    for cp in g_copies(k, b, b + 2 * q):
                        cp.wait()

            @pl.when(k >= 2)
            def _():
                o_copy(k - 2, b).wait()

            def row(r, acc_r):
                ss, qq = acc_r
                nss = []
                nqq = []
                for g in range(16):
                    sl = pl.ds(g * 16, 16)
                    y = bA[b][r, sl] + bB[b][r, sl] + bW[b][r, sl]
                    bY[b][r, sl] = y
                    nss.append(ss[g] + y)
                    nqq.append(qq[g] + y * y)
                return (tuple(nss), tuple(nqq))

            acc = lax.fori_loop(0, GCH, row, acc)
            o_copy(k, b).start()

            for q in range(2):
                @pl.when((k + 2 < nchunk) & (q2 == q))
                def _(q=q):
                    slot = b + 2 * (1 - q)
                    for cp in i_copies(k + 2, slot):
                        cp.wait()
                    for cp in g_copies(k + 2, b, slot):
                        cp.start()

                @pl.when((k + 4 < nchunk) & (q2 == q))
                def _(q=q):
                    for cp in i_copies(k + 4, b + 2 * q):
                        cp.start()
        return acc

    ss, qq = lax.fori_loop(0, nchunk // 2, dstep, acc0)
    o_copy(nchunk - 2, 0).wait()
    o_copy(nchunk - 1, 1).wait()
    for g in range(16):
        sl = pl.ds(g * 16, 16)
        accv[0, sl] = ss[g]
        accv[1, sl] = qq[g]
    pltpu.sync_copy(accv, mom.at[wid])


def _zero_spmem(zbuf, accS, s, rows_per_s, ncols):
    zrows = zbuf.shape[0]

    def zrow(r, _):
        for g in range(ncols // 16):
            zbuf[r, pl.ds(g * 16, 16)] = jnp.zeros((16,), F32)
        return 0
    lax.fori_loop(0, zrows, zrow, 0)
    for t in range(rows_per_s // zrows):
        pltpu.sync_copy(zbuf, accS.at[pl.ds(s * rows_per_s + t * zrows, zrows)])


def _copy_idx(dsta, srca, base, idxb, c):
    @pl.when(c == 0)
    def _():
        pltpu.sync_copy(dsta.at[pl.ds(base, CH)], idxb)

    @pl.when(c == 1)
    def _():
        pltpu.sync_copy(srca.at[pl.ds(base, CH)], idxb)


def _deg_body(ridx, out, idxc, oneb, zbuf, accS,
              isem0, isem1, ssem0, ssem1):
    NP = out.shape[1]
    nchunk = ridx.shape[2]
    rows_per_s = NP // NS
    c = lax.axis_index("c")
    s = lax.axis_index("s")
    isem = (isem0, isem1)
    ssem = (ssem0, ssem1)

    _zero_spmem(zbuf, accS, s, rows_per_s, 128)

    lane = lax.iota(jnp.int32, 16)
    onehot = jnp.where(lane == 0, jnp.float32(1.0), jnp.float32(0.0))
    zero16 = jnp.zeros((16,), F32)

    def prow(r, _):
        oneb[r, pl.ds(0, 16)] = onehot
        for g in range(1, 8):
            oneb[r, pl.ds(g * 16, 16)] = zero16
        return 0
    lax.fori_loop(0, CH, prow, 0)
    plsc.subcore_barrier()

    def i_copy(k, slot):
        return pltpu.make_async_copy(ridx.at[c, s, k, 0], idxc.at[slot],
                                     isem[slot % 2])

    def s_copy(k, slot):
        return pltpu.make_async_copy(oneb, accS.at[idxc.at[slot]],
                                     ssem[slot % 2])

    for q in (0, 1):
        i_copy(q, q).start()

    def dstep(k2, _):
        q2 = k2 % 2
        for b in range(2):
            k = 2 * k2 + b
            for q in range(2):
                @pl.when(q2 == q)
                def _(q=q):
                    slot = b + 2 * q
                    other = b + 2 * (1 - q)
                    i_copy(k, slot).wait()

                    @pl.when(k >= 2)
                    def _():
                        s_copy(k - 2, other).wait()

                    s_copy(k, slot).start(add=True)

                    @pl.when(k + 2 < nchunk)
                    def _():
                        i_copy(k + 2, other).start()
        return 0

    lax.fori_loop(0, nchunk // 2, dstep, 0)
    s_copy(nchunk - 2, (nchunk - 2) % 4).wait()
    s_copy(nchunk - 1, (nchunk - 1) % 4).wait()
    plsc.subcore_barrier()
    pltpu.sync_copy(accS.at[pl.ds(s * rows_per_s, rows_per_s)],
                    out.at[c, pl.ds(s * rows_per_s, rows_per_s)])


def _scatter_body(y2, ridx, ac2, out,
                  idxc, y2b, msgb, zbuf, acv, accS,
                  ysem0, ysem1, ssem0, ssem1):
    NP = out.shape[1]
    nchunk = ridx.shape[2]
    per_s = nchunk * CH
    rows_per_s = NP // NS
    c = lax.axis_index("c")
    s = lax.axis_index("s")

    yb = (y2b.at[0], y2b.at[1])
    mb = (msgb.at[0], msgb.at[1])
    ysem = (ysem0, ysem1)
    ssem = (ssem0, ssem1)

    pltpu.sync_copy(ac2.at[c], acv)
    _zero_spmem(zbuf, accS, s, rows_per_s, 128)
    plsc.subcore_barrier()

    def y_copies(k, b, slot):
        base = s * per_s + k * CH
        return (
            pltpu.make_async_copy(y2.at[c, pl.ds(base, CH)], yb[b], ysem[b]),
            pltpu.make_async_copy(ridx.at[c, s, k, 0], idxc.at[slot], ysem[b]),
        )

    def s_copy(k, b, slot):
        return pltpu.make_async_copy(mb[b], accS.at[idxc.at[slot]], ssem[b])

    for q in (0, 1):
        for cp in y_copies(q, q, q):
            cp.start()

    def dstep(k2, _):
        q2 = k2 % 2
        for b in range(2):
            k = 2 * k2 + b
            for q in range(2):
                @pl.when(q2 == q)
                def _(q=q):
                    for cp in y_copies(k, b, b + 2 * q):
                        cp.wait()

                @pl.when((k >= 2) & (q2 == q))
                def _(q=q):
                    s_copy(k - 2, b, b + 2 * (1 - q)).wait()

            def row(r, __):
                for g in range(8):
                    sl = pl.ds(g * 16, 16)
                    v = acv[0, sl] * yb[b][r, sl] + acv[1, sl]
                    mb[b][r, sl] = jnp.maximum(v, 0.0)
                return 0
            lax.fori_loop(0, CH, row, 0)

            for q in range(2):
                @pl.when(q2 == q)
                def _(q=q):
                    s_copy(k, b, b + 2 * q).start(add=True)

                @pl.when((k + 2 < nchunk) & (q2 == q))
                def _(q=q):
                    for cp in y_copies(k + 2, b, b + 2 * (1 - q)):
                        cp.start()
        return 0

    lax.fori_loop(0, nchunk // 2, dstep, 0)
    s_copy(nchunk - 2, 0, (nchunk - 2) % 4).wait()
    s_copy(nchunk - 1, 1, (nchunk - 1) % 4).wait()
    plsc.subcore_barrier()
    pltpu.sync_copy(accS.at[pl.ds(s * rows_per_s, rows_per_s)],
                    out.at[c, pl.ds(s * rows_per_s, rows_per_s)])


def kernel(nodes, edges, edge_attr,
           pW1, pb1, pg1, pbt1, pW2, pb2, pg2, pbt2,
           cW1, cb1, cg1, cbt1, cW2, cb2, cg2, cbt2,
           fW, fb, fg, fbt):
    N, D = nodes.shape
    M = edges.shape[1]
    E = edge_attr.shape[1]
    D2 = 2 * D

    Wa = jnp.concatenate([pW1[:D], cW1[D:D2]], axis=1)
    Wb = jnp.concatenate([pW1[D:D2], cW1[:D]], axis=1)
    We = jnp.concatenate([pW1[D2:], cW1[D2:]], axis=1)
    be = jnp.concatenate([pb1, cb1]).reshape(1, D2)
    g1v = jnp.concatenate([pg1, cg1])
    bt1v = jnp.concatenate([pbt1, cbt1])
    b2s = jnp.stack([pb2, cb2])
    g2s = jnp.stack([pg2, cg2])
    bt2s = jnp.stack([pbt2, cbt2])
    fp = jnp.stack([fb, fg, fbt])

    BLK_E = 2000
    w = pl.pallas_call(
        _edge_w_body,
        grid=(M // BLK_E,),
        in_specs=[
            pl.BlockSpec((BLK_E, E), lambda i: (i, 0)),
            pl.BlockSpec((E, D2), lambda i: (0, 0)),
            pl.BlockSpec((1, D2), lambda i: (0, 0)),
        ],
        out_specs=pl.BlockSpec((BLK_E, D2), lambda i: (i, 0)),
        out_shape=jax.ShapeDtypeStruct((M, D2), F32),
    )(edge_attr, We, be)

    ab_call = pl.pallas_call(
        _ab_body,
        out_shape=(jax.ShapeDtypeStruct((N, D2), F32),
                   jax.ShapeDtypeStruct((N, D2), F32)),
    )

    GN = M // NW // GCH
    gather_call = pl.kernel(
        _gather_body,
        out_type=(jax.ShapeDtypeStruct((M, D2), F32),
                  jax.ShapeDtypeStruct((NW, 2, D2), F32)),
        mesh=_SC_MESH,
        scratch_types=[
            pltpu.VMEM((4, GCH), jnp.int32),
            pltpu.VMEM((4, GCH), jnp.int32),
            pltpu.VMEM((2, GCH, D2), F32),
            pltpu.VMEM((2, GCH, D2), F32),
            pltpu.VMEM((2, GCH, D2), F32),
            pltpu.VMEM((2, GCH, D2), F32),
            pltpu.VMEM((2, D2), F32),
            pltpu.SemaphoreType.DMA,
            pltpu.SemaphoreType.DMA,
            pltpu.SemaphoreType.DMA,
            pltpu.SemaphoreType.DMA,
            pltpu.SemaphoreType.DMA,
            pltpu.SemaphoreType.DMA,
        ],
    )

    BLK_M = 512
    mid_call = pl.pallas_call(
        functools.partial(_mid_body, M),
        grid=(M // BLK_M,),
        in_specs=[
            pl.BlockSpec((BLK_M, D2), lambda i: (i, 0)),
            pl.BlockSpec((NW, 2, D2), lambda i: (0, 0, 0)),
            pl.BlockSpec((2, D2), lambda i: (0, 0)),
            pl.BlockSpec((D, D), lambda i: (0, 0)),
            pl.BlockSpec((D, D), lambda i: (0, 0)),
            pl.BlockSpec((2, D), lambda i: (0, 0)),
            pl.BlockSpec((4, D), lambda i: (0, 0)),
        ],
        out_specs=(pl.BlockSpec((2, BLK_M, D), lambda i: (0, i, 0)),
                   pl.BlockSpec((2, 2, D), lambda i: (0, 0, 0))),
        out_shape=(jax.ShapeDtypeStruct((2, M, D), F32),
                   jax.ShapeDtypeStruct((2, 2, D), F32)),
        scratch_shapes=[pltpu.VMEM((2, D2), F32), pltpu.VMEM((4, D), F32)],
    )

    NP = ((N + 128 * NS - 1) // (128 * NS)) * (128 * NS)
    SN = M // NS // CH
    scatter_call = pl.kernel(
        _scatter_body,
        out_type=jax.ShapeDtypeStruct((2, NP, D), F32),
        mesh=_SC_MESH,
        scratch_types=[
            pltpu.VMEM((4, CH), jnp.int32),
            pltpu.VMEM((2, CH, D), F32),
            pltpu.VMEM((2, CH, D), F32),
            pltpu.VMEM((32, D), F32),
            pltpu.VMEM((2, D), F32),
            pltpu.VMEM_SHARED((NP, D), F32),
            pltpu.SemaphoreType.DMA,
            pltpu.SemaphoreType.DMA,
            pltpu.SemaphoreType.DMA,
            pltpu.SemaphoreType.DMA,
        ],
    )

    deg_call = pl.kernel(
        _deg_body,
        out_type=jax.ShapeDtypeStruct((2, NP, D), F32),
        mesh=_SC_MESH,
        scratch_types=[
            pltpu.VMEM((4, CH), jnp.int32),
            pltpu.VMEM((CH, D), F32),
            pltpu.VMEM((128, D), F32),
            pltpu.VMEM_SHARED((NP, D), F32),
            pltpu.SemaphoreType.DMA,
            pltpu.SemaphoreType.DMA,
            pltpu.SemaphoreType.DMA,
            pltpu.SemaphoreType.DMA,
        ],
    )

    final_call = pl.pallas_call(
        _final_body,
        out_shape=(jax.ShapeDtypeStruct((N, D), F32),
                   jax.ShapeDtypeStruct((N, D2), F32),
                   jax.ShapeDtypeStruct((N, D2), F32)),
    )

    gbt1 = jnp.stack([g1v, bt1v])
    gbt2 = jnp.concatenate([g2s, bt2s])
    pW2b = pW2.astype(jnp.bfloat16)
    cW2b = cW2.astype(jnp.bfloat16)
    dsta = edges[1]
    srca = edges[0]
    dst3g = dsta.reshape(NW, GN, 1, GCH)
    src3g = srca.reshape(NW, GN, 1, GCH)
    ridx = jnp.stack([dsta, srca]).reshape(2, NS, SN, 1, CH)
    degs = deg_call(ridx)
    x = nodes
    A, B = ab_call(x, Wa, Wb)
    for _ in range(2):
        y1, mom = gather_call(A, B, dst3g, src3g, w)
        y2, ac2 = mid_call(y1, mom, gbt1, pW2b, cW2b, b2s, gbt2)
        agg = scatter_call(y2, ridx, ac2)
        x, A, B = final_call(x, agg, degs, fW, fp, Wa, Wb)
    return x


# speedup vs baseline: 2.3095x; 1.8634x over previous
import functools

import jax
import jax.numpy as jnp
from jax import lax
from jax.experimental import pallas as pl
from jax.experimental.pallas import tpu as pltpu
from jax.experimental.pallas import tpu_sc as plsc

NC = 2
NS = 16
NW = NC * NS
CH = 80
EPS = 1e-5
F32 = jnp.float32

_SC_MESH = plsc.VectorSubcoreMesh(
    core_axis_name="c", subcore_axis_name="s", num_cores=NC, num_subcores=NS)


def _edge_w_body(e_ref, W_ref, b_ref, o_ref):
    o_ref[...] = (
        jnp.dot(e_ref[...], W_ref[...], preferred_element_type=F32)
        + b_ref[...])


def _ab_body(x_ref, Wa_ref, Wb_ref, a_ref, b_ref):
    a_ref[...] = jnp.dot(x_ref[...], Wa_ref[...], preferred_element_type=F32)
    b_ref[...] = jnp.dot(x_ref[...], Wb_ref[...], preferred_element_type=F32)


def _mid_body(M, y1_ref, mom_ref, gbt1_ref, Wp_ref, Wc_ref, b2_ref, gbt2_ref,
              y2_ref, ac2_ref, a1c1_ref, macc_ref):
    D = Wp_ref.shape[0]

    @pl.when(pl.program_id(0) == 0)
    def _init():
        msum = jnp.sum(mom_ref[...], axis=0)
        mean1 = msum[0:1] / M
        var1 = msum[1:2] / M - mean1 * mean1
        a1 = gbt1_ref[0:1] * jax.lax.rsqrt(var1 + EPS)
        a1c1_ref[0:1] = a1
        a1c1_ref[1:2] = gbt1_ref[1:2] - a1 * mean1
        macc_ref[...] = jnp.zeros_like(macc_ref)
        ac2_ref[...] = jnp.zeros_like(ac2_ref)

    h = jnp.maximum(y1_ref[...] * a1c1_ref[0:1] + a1c1_ref[1:2], 0.0)
    hb = h.astype(jnp.bfloat16)
    y2p = jnp.dot(hb[:, :D], Wp_ref[...], preferred_element_type=F32) + b2_ref[0:1, :]
    y2c = jnp.dot(hb[:, D:], Wc_ref[...], preferred_element_type=F32) + b2_ref[1:2, :]
    y2_ref[0] = y2p
    y2_ref[1] = y2c

    macc_ref[0:1] += jnp.sum(y2p, axis=0, keepdims=True)
    macc_ref[1:2] += jnp.sum(y2p * y2p, axis=0, keepdims=True)
    macc_ref[2:3] += jnp.sum(y2c, axis=0, keepdims=True)
    macc_ref[3:4] += jnp.sum(y2c * y2c, axis=0, keepdims=True)

    @pl.when(pl.program_id(0) == pl.num_programs(0) - 1)
    def _fin():
        mean2 = jnp.stack([macc_ref[0], macc_ref[2]]) / M
        var2 = jnp.stack([macc_ref[1], macc_ref[3]]) / M - mean2 * mean2
        a2 = gbt2_ref[0:2] * jax.lax.rsqrt(var2 + EPS)
        c2 = gbt2_ref[2:4] - a2 * mean2
        ac2_ref[...] = jnp.stack([a2, c2], axis=1)


def _final_body(x_ref, agg_ref, deg_ref, fW_ref, fp_ref, Wa_ref, Wb_ref,
                xo_ref, a_ref, b_ref):
    N, D = x_ref.shape
    op = agg_ref[0, :N]
    oc = agg_ref[1, :N]
    deg_d = deg_ref[0, :N, 0:1]
    deg_s = deg_ref[1, :N, 0:1]
    fi = jnp.where(deg_d > 0, 1.0 / deg_d, 0.0) * op
    fo = jnp.where(deg_s > 0, 1.0 / deg_s, 0.0) * oc
    z = x_ref[...] + fi + fo
    y = jnp.dot(z, fW_ref[...], preferred_element_type=F32) + fp_ref[0:1, :]
    mu = jnp.mean(y, axis=0, keepdims=True)
    var = jnp.mean((y - mu) * (y - mu), axis=0, keepdims=True)
    bn = fp_ref[1:2, :] * (y - mu) / jnp.sqrt(var + EPS) + fp_ref[2:3, :]
    xn = x_ref[...] + jnp.maximum(bn, 0.0)
    xo_ref[...] = xn
    a_ref[...] = jnp.dot(xn, Wa_ref[...], preferred_element_type=F32)
    b_ref[...] = jnp.dot(xn, Wb_ref[...], preferred_element_type=F32)


GCH = 40


def _gather_body(A, B, dst3, src3, w, y1, mom,
                 idxd, idxs, bufA, bufB, bufW, bufY, accv,
                 gsem0, gsem1, osem0, osem1, isem0, isem1):
    nchunk = dst3.shape[1]
    per_w = nchunk * GCH
    c = lax.axis_index("c")
    s = lax.axis_index("s")
    wid = s * NC + c
    base_w = wid * per_w

    bA = (bufA.at[0], bufA.at[1])
    bB = (bufB.at[0], bufB.at[1])
    bW = (bufW.at[0], bufW.at[1])
    bY = (bufY.at[0], bufY.at[1])
    gsem = (gsem0, gsem1)
    osem = (osem0, osem1)
    isem = (isem0, isem1)

    def i_copies(k, slot):
        sem = isem[slot % 2]
        return (
            pltpu.make_async_copy(dst3.at[wid, k, 0], idxd.at[slot], sem),
            pltpu.make_async_copy(src3.at[wid, k, 0], idxs.at[slot], sem),
        )

    def g_copies(k, b, slot):
        base = base_w + k * GCH
        return (
            pltpu.make_async_copy(A.at[idxd.at[slot]], bA[b], gsem[b]),
            pltpu.make_async_copy(B.at[idxs.at[slot]], bB[b], gsem[b]),
            pltpu.make_async_copy(w.at[pl.ds(base, GCH)], bW[b], gsem[b]),
        )

    def o_copy(k, b):
        base = base_w + k * GCH
        return pltpu.make_async_copy(bY[b], y1.at[pl.ds(base, GCH)], osem[b])

    for q in (0, 1):
        for cp in i_copies(q, q):
            cp.start()
        for cp in i_copies(q, q):
            cp.wait()
    for q in (2, 3):
        for cp in i_copies(q, q):
            cp.start()
    for cp in g_copies(0, 0, 0):
        cp.start()
    for cp in g_copies(1, 1, 1):
        cp.start()

    zero16 = jnp.zeros((16,), F32)
    acc0 = (tuple(zero16 for _ in range(16)), tuple(zero16 for _ in range(16)))

    def dstep(k2, acc):
        q2 = k2 % 2
        for b in range(2):
            k = 2 * k2 + b
            for q in range(2):
                @pl.when(q2 == q)
                def _(q=q):
                    for cp in g_copies(k, b, b + 2 * q):
                        cp.wait()

            @pl.when(k >= 2)
            def _():
                o_copy(k - 2, b).wait()

            def row(r, acc_r):
                ss, qq = acc_r
                nss = []
                nqq = []
                for g in range(16):
                    sl = pl.ds(g * 16, 16)
                    y = bA[b][r, sl] + bB[b][r, sl] + bW[b][r, sl]
                    bY[b][r, sl] = y
                    nss.append(ss[g] + y)
                    nqq.append(qq[g] + y * y)
                return (tuple(nss), tuple(nqq))

            acc = lax.fori_loop(0, GCH, row, acc)
            o_copy(k, b).start()

            for q in range(2):
                @pl.when((k + 2 < nchunk) & (q2 == q))
                def _(q=q):
                    slot = b + 2 * (1 - q)
                    for cp in i_copies(k + 2, slot):
                        cp.wait()
                    for cp in g_copies(k + 2, b, slot):
                        cp.start()

                @pl.when((k + 4 < nchunk) & (q2 == q))
                def _(q=q):
                    for cp in i_copies(k + 4, b + 2 * q):
                        cp.start()
        return acc

    ss, qq = lax.fori_loop(0, nchunk // 2, dstep, acc0)
    o_copy(nchunk - 2, 0).wait()
    o_copy(nchunk - 1, 1).wait()
    for g in range(16):
        sl = pl.ds(g * 16, 16)
        accv[0, sl] = ss[g]
        accv[1, sl] = qq[g]
    pltpu.sync_copy(accv, mom.at[wid])


def _zero_spmem(zbuf, accS, s, rows_per_s, ncols):
    zrows = zbuf.shape[0]

    def zrow(r, _):
        for g in range(ncols // 16):
            zbuf[r, pl.ds(g * 16, 16)] = jnp.zeros((16,), F32)
        return 0
    lax.fori_loop(0, zrows, zrow, 0)
    for t in range(rows_per_s // zrows):
        pltpu.sync_copy(zbuf, accS.at[pl.ds(s * rows_per_s + t * zrows, zrows)])


def _copy_idx(dsta, srca, base, idxb, c):
    @pl.when(c == 0)
    def _():
        pltpu.sync_copy(dsta.at[pl.ds(base, CH)], idxb)

    @pl.when(c == 1)
    def _():
        pltpu.sync_copy(srca.at[pl.ds(base, CH)], idxb)


def _deg_body(ridx, out, idxc, oneb, zbuf, accS,
              isem0, isem1, ssem0, ssem1):
    NP = out.shape[1]
    nchunk = ridx.shape[2]
    rows_per_s = NP // NS
    c = lax.axis_index("c")
    s = lax.axis_index("s")
    isem = (isem0, isem1)
    ssem = (ssem0, ssem1)

    _zero_spmem(zbuf, accS, s, rows_per_s, 128)

    lane = lax.iota(jnp.int32, 16)
    onehot = jnp.where(lane == 0, jnp.float32(1.0), jnp.float32(0.0))
    zero16 = jnp.zeros((16,), F32)

    def prow(r, _):
        oneb[r, pl.ds(0, 16)] = onehot
        for g in range(1, 8):
            oneb[r, pl.ds(g * 16, 16)] = zero16
        return 0
    lax.fori_loop(0, CH, prow, 0)
    plsc.subcore_barrier()

    def i_copy(k, slot):
        return pltpu.make_async_copy(ridx.at[c, s, k, 0], idxc.at[slot],
                                     isem[slot % 2])

    def s_copy(k, slot):
        return pltpu.make_async_copy(oneb, accS.at[idxc.at[slot]],
                                     ssem[slot % 2])

    for q in (0, 1):
        i_copy(q, q).start()

    def dstep(k2, _):
        q2 = k2 % 2
        for b in range(2):
            k = 2 * k2 + b
            for q in range(2):
                @pl.when(q2 == q)
                def _(q=q):
                    slot = b + 2 * q
                    other = b + 2 * (1 - q)
                    i_copy(k, slot).wait()

                    @pl.when(k >= 2)
                    def _():
                        s_copy(k - 2, other).wait()

                    s_copy(k, slot).start(add=True)

                    @pl.when(k + 2 < nchunk)
                    def _():
                        i_copy(k + 2, other).start()
        return 0

    lax.fori_loop(0, nchunk // 2, dstep, 0)
    s_copy(nchunk - 2, (nchunk - 2) % 4).wait()
    s_copy(nchunk - 1, (nchunk - 1) % 4).wait()
    plsc.subcore_barrier()
    pltpu.sync_copy(accS.at[pl.ds(s * rows_per_s, rows_per_s)],
                    out.at[c, pl.ds(s * rows_per_s, rows_per_s)])


def _scatter_body(y2, ridx, ac2, out,
                  idxc, y2b, msgb, zbuf, acv, accS,
                  ysem0, ysem1, ssem0, ssem1):
    NP = out.shape[1]
    nchunk = ridx.shape[2]
    per_s = nchunk * CH
    rows_per_s = NP // NS
    c = lax.axis_index("c")
    s = lax.axis_index("s")

    yb = (y2b.at[0], y2b.at[1])
    mb = (msgb.at[0], msgb.at[1])
    ysem = (ysem0, ysem1)
    ssem = (ssem0, ssem1)

    pltpu.sync_copy(ac2.at[c], acv)
    _zero_spmem(zbuf, accS, s, rows_per_s, 128)
    plsc.subcore_barrier()

    def y_copies(k, b, slot):
        base = s * per_s + k * CH
        return (
            pltpu.make_async_copy(y2.at[c, pl.ds(base, CH)], yb[b], ysem[b]),
            pltpu.make_async_copy(ridx.at[c, s, k, 0], idxc.at[slot], ysem[b]),
        )

    def s_copy(k, b, slot):
        return pltpu.make_async_copy(mb[b], accS.at[idxc.at[slot]], ssem[b])

    for q in (0, 1):
        for cp in y_copies(q, q, q):
            cp.start()

    def dstep(k2, _):
        q2 = k2 % 2
        for b in range(2):
            k = 2 * k2 + b
            for q in range(2):
                @pl.when(q2 == q)
                def _(q=q):
                    for cp in y_copies(k, b, b + 2 * q):
                        cp.wait()

                @pl.when((k >= 2) & (q2 == q))
                def _(q=q):
                    s_copy(k - 2, b, b + 2 * (1 - q)).wait()

            @functools.partial(plsc.parallel_loop, 0, CH, unroll=2)
            def row(r):
                for g in range(8):
                    sl = pl.ds(g * 16, 16)
                    v = acv[0, sl] * yb[b][r, sl] + acv[1, sl]
                    mb[b][r, sl] = jnp.maximum(v, 0.0)

            for q in range(2):
                @pl.when(q2 == q)
                def _(q=q):
                    s_copy(k, b, b + 2 * q).start(add=True)

                @pl.when((k + 2 < nchunk) & (q2 == q))
                def _(q=q):
                    for cp in y_copies(k + 2, b, b + 2 * (1 - q)):
                        cp.start()
        return 0

    lax.fori_loop(0, nchunk // 2, dstep, 0)
    s_copy(nchunk - 2, 0, (nchunk - 2) % 4).wait()
    s_copy(nchunk - 1, 1, (nchunk - 1) % 4).wait()
    plsc.subcore_barrier()
    pltpu.sync_copy(accS.at[pl.ds(s * rows_per_s, rows_per_s)],
                    out.at[c, pl.ds(s * rows_per_s, rows_per_s)])


def kernel(nodes, edges, edge_attr,
           pW1, pb1, pg1, pbt1, pW2, pb2, pg2, pbt2,
           cW1, cb1, cg1, cbt1, cW2, cb2, cg2, cbt2,
           fW, fb, fg, fbt):
    N, D = nodes.shape
    M = edges.shape[1]
    E = edge_attr.shape[1]
    D2 = 2 * D

    Wa = jnp.concatenate([pW1[:D], cW1[D:D2]], axis=1)
    Wb = jnp.concatenate([pW1[D:D2], cW1[:D]], axis=1)
    We = jnp.concatenate([pW1[D2:], cW1[D2:]], axis=1)
    be = jnp.concatenate([pb1, cb1]).reshape(1, D2)
    g1v = jnp.concatenate([pg1, cg1])
    bt1v = jnp.concatenate([pbt1, cbt1])
    b2s = jnp.stack([pb2, cb2])
    g2s = jnp.stack([pg2, cg2])
    bt2s = jnp.stack([pbt2, cbt2])
    fp = jnp.stack([fb, fg, fbt])

    BLK_E = 2000
    w = pl.pallas_call(
        _edge_w_body,
        grid=(M // BLK_E,),
        in_specs=[
            pl.BlockSpec((BLK_E, E), lambda i: (i, 0)),
            pl.BlockSpec((E, D2), lambda i: (0, 0)),
            pl.BlockSpec((1, D2), lambda i: (0, 0)),
        ],
        out_specs=pl.BlockSpec((BLK_E, D2), lambda i: (i, 0)),
        out_shape=jax.ShapeDtypeStruct((M, D2), F32),
    )(edge_attr, We, be)

    ab_call = pl.pallas_call(
        _ab_body,
        out_shape=(jax.ShapeDtypeStruct((N, D2), F32),
                   jax.ShapeDtypeStruct((N, D2), F32)),
    )

    GN = M // NW // GCH
    gather_call = pl.kernel(
        _gather_body,
        out_type=(jax.ShapeDtypeStruct((M, D2), F32),
                  jax.ShapeDtypeStruct((NW, 2, D2), F32)),
        mesh=_SC_MESH,
        scratch_types=[
            pltpu.VMEM((4, GCH), jnp.int32),
            pltpu.VMEM((4, GCH), jnp.int32),
            pltpu.VMEM((2, GCH, D2), F32),
            pltpu.VMEM((2, GCH, D2), F32),
            pltpu.VMEM((2, GCH, D2), F32),
            pltpu.VMEM((2, GCH, D2), F32),
            pltpu.VMEM((2, D2), F32),
            pltpu.SemaphoreType.DMA,
            pltpu.SemaphoreType.DMA,
            pltpu.SemaphoreType.DMA,
            pltpu.SemaphoreType.DMA,
            pltpu.SemaphoreType.DMA,
            pltpu.SemaphoreType.DMA,
        ],
    )

    BLK_M = 1280
    mid_call = pl.pallas_call(
        functools.partial(_mid_body, M),
        grid=(M // BLK_M,),
        in_specs=[
            pl.BlockSpec((BLK_M, D2), lambda i: (i, 0)),
            pl.BlockSpec((NW, 2, D2), lambda i: (0, 0, 0)),
            pl.BlockSpec((2, D2), lambda i: (0, 0)),
            pl.BlockSpec((D, D), lambda i: (0, 0)),
            pl.BlockSpec((D, D), lambda i: (0, 0)),
            pl.BlockSpec((2, D), lambda i: (0, 0)),
            pl.BlockSpec((4, D), lambda i: (0, 0)),
        ],
        out_specs=(pl.BlockSpec((2, BLK_M, D), lambda i: (0, i, 0)),
                   pl.BlockSpec((2, 2, D), lambda i: (0, 0, 0))),
        out_shape=(jax.ShapeDtypeStruct((2, M, D), F32),
                   jax.ShapeDtypeStruct((2, 2, D), F32)),
        scratch_shapes=[pltpu.VMEM((2, D2), F32), pltpu.VMEM((4, D), F32)],
    )

    NP = ((N + 128 * NS - 1) // (128 * NS)) * (128 * NS)
    SN = M // NS // CH
    scatter_call = pl.kernel(
        _scatter_body,
        out_type=jax.ShapeDtypeStruct((2, NP, D), F32),
        mesh=_SC_MESH,
        scratch_types=[
            pltpu.VMEM((4, CH), jnp.int32),
            pltpu.VMEM((2, CH, D), F32),
            pltpu.VMEM((2, CH, D), F32),
            pltpu.VMEM((32, D), F32),
            pltpu.VMEM((2, D), F32),
            pltpu.VMEM_SHARED((NP, D), F32),
            pltpu.SemaphoreType.DMA,
            pltpu.SemaphoreType.DMA,
            pltpu.SemaphoreType.DMA,
            pltpu.SemaphoreType.DMA,
        ],
    )

    deg_call = pl.kernel(
        _deg_body,
        out_type=jax.ShapeDtypeStruct((2, NP, D), F32),
        mesh=_SC_MESH,
        scratch_types=[
            pltpu.VMEM((4, CH), jnp.int32),
            pltpu.VMEM((CH, D), F32),
            pltpu.VMEM((128, D), F32),
            pltpu.VMEM_SHARED((NP, D), F32),
            pltpu.SemaphoreType.DMA,
            pltpu.SemaphoreType.DMA,
            pltpu.SemaphoreType.DMA,
            pltpu.SemaphoreType.DMA,
        ],
    )

    final_call = pl.pallas_call(
        _final_body,
        out_shape=(jax.ShapeDtypeStruct((N, D), F32),
                   jax.ShapeDtypeStruct((N, D2), F32),
                   jax.ShapeDtypeStruct((N, D2), F32)),
    )

    gbt1 = jnp.stack([g1v, bt1v])
    gbt2 = jnp.concatenate([g2s, bt2s])
    pW2b = pW2.astype(jnp.bfloat16)
    cW2b = cW2.astype(jnp.bfloat16)
    dsta = edges[1]
    srca = edges[0]
    dst3g = dsta.reshape(NW, GN, 1, GCH)
    src3g = srca.reshape(NW, GN, 1, GCH)
    ridx = jnp.stack([dsta, srca]).reshape(2, NS, SN, 1, CH)
    degs = deg_call(ridx)
    x = nodes
    A, B = ab_call(x, Wa, Wb)
    for _ in range(2):
        y1, mom = gather_call(A, B, dst3g, src3g, w)
        y2, ac2 = mid_call(y1, mom, gbt1, pW2b, cW2b, b2s, gbt2)
        agg = scatter_call(y2, ridx, ac2)
        x, A, B = final_call(x, agg, degs, fW, fp, Wa, Wb)
    return x
